# baseline (speedup 1.0000x reference)
import functools

import jax
import jax.numpy as jnp
from jax import lax
from jax.experimental import pallas as pl
from jax.experimental.pallas import tpu as pltpu
from jax.experimental.pallas import tpu_sc as plsc

B, Y, X, BINS = 128, 96, 96, 64
KP = 17
Z_SIZE = 1.0
NROWS = B * KP
NC, NS, L = 2, 16, 16
NW = NC * NS
RPW = NROWS // NW
RPAD = 80
OPAD = 72


def _tec_body(feat_hbm, pidx_hbm, pose_out, prob_out,
              pidx_v, gidx_v, rows_v, probs_v, poses_v, sem):
    wid = lax.axis_index("s") * NC + lax.axis_index("c")
    pltpu.sync_copy(pidx_hbm.at[wid], pidx_v)

    lane = lax.iota(jnp.int32, L)
    for c in range(RPAD // L):
        bi = pidx_v[0, pl.ds(c * L, L)]
        yi = pidx_v[1, pl.ds(c * L, L)]
        xi = pidx_v[2, pl.ds(c * L, L)]
        gidx_v[pl.ds(c * L, L)] = bi * (Y * X) + yi * X + xi

    pltpu.async_copy(feat_hbm.at[gidx_v], rows_v, sem).wait()

    locs = [(lax.iota(jnp.int32, L) + k * L).astype(jnp.float32)
            * (2.0 * Z_SIZE / (BINS - 1)) - Z_SIZE for k in range(BINS // L)]

    def group_body(g, carry):
        acc = jnp.zeros((L,), jnp.float32)
        for j in range(L):
            r = g * L + j
            vs = [rows_v[r, pl.ds(k * L, L)] for k in range(BINS // L)]
            m = jnp.max(jnp.maximum(jnp.maximum(vs[0], vs[1]),
                                    jnp.maximum(vs[2], vs[3])))
            es = [jnp.exp(v - m) for v in vs]
            s = jnp.sum(es[0] + es[1] + es[2] + es[3])
            w = jnp.sum(es[0] * locs[0] + es[1] * locs[1]
                        + es[2] * locs[2] + es[3] * locs[3])
            invv = 1.0 / lax.broadcast_in_dim(s, (L,), ())
            for k in range(BINS // L):
                probs_v[r, pl.ds(k * L, L)] = es[k] * invv
            acc = jnp.where(lane == j,
                            lax.broadcast_in_dim(w, (L,), ()) * invv, acc)
        poses_v[pl.ds(pl.multiple_of(g * L, L), L)] = acc
        return carry

    lax.fori_loop(0, RPAD // L, group_body, 0)

    pltpu.sync_copy(poses_v.at[pl.ds(0, OPAD)],
                    pose_out.at[pl.ds(wid * OPAD, OPAD)])
    pltpu.sync_copy(probs_v.at[pl.ds(0, OPAD)],
                    prob_out.at[pl.ds(wid * OPAD, OPAD)])


@functools.partial(jax.jit)
def _sc_extract(feat, pidx):
    run = functools.partial(
        pl.kernel,
        out_type=[
            jax.ShapeDtypeStruct((NW * OPAD,), jnp.float32),
            jax.ShapeDtypeStruct((NW * OPAD, BINS), jnp.float32),
        ],
        mesh=plsc.VectorSubcoreMesh(core_axis_name="c", subcore_axis_name="s"),
        compiler_params=pltpu.CompilerParams(
            needs_layout_passes=False, use_tc_tiling_on_sc=False),
        scratch_types=[
            pltpu.VMEM((3, RPAD), jnp.int32),
            pltpu.VMEM((RPAD,), jnp.int32),
            pltpu.VMEM((RPAD, BINS), jnp.float32),
            pltpu.VMEM((RPAD, BINS), jnp.float32),
            pltpu.VMEM((RPAD,), jnp.float32),
            pltpu.SemaphoreType.DMA,
        ],
    )(_tec_body)
    return run(feat, pidx)


def kernel(features_z, pose_indexes):
    feat = features_z.reshape(B * Y * X, BINS)
    pidx = jnp.pad(
        pose_indexes.reshape(NW, RPW, 3).transpose(0, 2, 1),
        ((0, 0), (0, 0), (0, RPAD - RPW)))
    poses_pad, probs_pad = _sc_extract(feat, pidx)
    poses = poses_pad.reshape(NW, OPAD)[:, :RPW].reshape(B, KP)
    probs = probs_pad.reshape(NW, OPAD, BINS)[:, :RPW].reshape(B, KP, BINS)
    return poses, probs


# speedup vs baseline: 2.5027x; 2.5027x over previous
import functools

import jax
import jax.numpy as jnp
from jax import lax
from jax.experimental import pallas as pl
from jax.experimental.pallas import tpu as pltpu
from jax.experimental.pallas import tpu_sc as plsc

B, Y, X, BINS = 128, 96, 96, 64
KP = 17
Z_SIZE = 1.0
NROWS = B * KP
NC, NS, L = 2, 16, 16
NW = NC * NS
RPW = NROWS // NW
RPAD = 80
OPAD = 72


def _tec_body(feat_hbm, b_hbm, y_hbm, x_hbm, pose_out, prob_out,
              b_v, y_v, x_v, rows_v, probs_v, poses_v, sem):
    wid = lax.axis_index("s") * NC + lax.axis_index("c")
    base = wid * RPAD
    pltpu.sync_copy(b_hbm.at[pl.ds(base, RPAD)], b_v)
    pltpu.sync_copy(y_hbm.at[pl.ds(base, RPAD)], y_v)
    pltpu.sync_copy(x_hbm.at[pl.ds(base, RPAD)], x_v)

    lane = lax.iota(jnp.int32, L)
    copies = []
    for c in range(RPAD // L):
        bi = b_v[pl.ds(c * L, L)]
        yi = y_v[pl.ds(c * L, L)]
        xi = x_v[pl.ds(c * L, L)]
        fi = bi * (Y * X) + yi * X + xi
        for j in range(L):
            copies.append(pltpu.async_copy(
                feat_hbm.at[fi[j]], rows_v.at[c * L + j], sem))
    for cp in copies:
        cp.wait()

    locs = [(lax.iota(jnp.int32, L) + k * L).astype(jnp.float32)
            * (2.0 * Z_SIZE / (BINS - 1)) - Z_SIZE for k in range(BINS // L)]

    def group_body(g, carry):
        acc = jnp.zeros((L,), jnp.float32)
        for j in range(L):
            r = g * L + j
            vs = [rows_v[r, pl.ds(k * L, L)] for k in range(BINS // L)]
            m = jnp.max(jnp.maximum(jnp.maximum(vs[0], vs[1]),
                                    jnp.maximum(vs[2], vs[3])))
            es = [jnp.exp(v - m) for v in vs]
            s = jnp.sum(es[0] + es[1] + es[2] + es[3])
            w = jnp.sum(es[0] * locs[0] + es[1] * locs[1]
                        + es[2] * locs[2] + es[3] * locs[3])
            invv = 1.0 / lax.broadcast_in_dim(s, (L,), ())
            for k in range(BINS // L):
                probs_v[r, pl.ds(k * L, L)] = es[k] * invv
            acc = jnp.where(lane == j,
                            lax.broadcast_in_dim(w, (L,), ()) * invv, acc)
        poses_v[pl.ds(pl.multiple_of(g * L, L), L)] = acc
        return carry

    lax.fori_loop(0, RPAD // L, group_body, 0)

    pltpu.sync_copy(poses_v.at[pl.ds(0, OPAD)],
                    pose_out.at[pl.ds(wid * OPAD, OPAD)])
    pltpu.sync_copy(probs_v.at[pl.ds(0, OPAD)],
                    prob_out.at[pl.ds(wid * OPAD, OPAD)])


@functools.partial(jax.jit)
def _sc_extract(feat, b1d, y1d, x1d):
    run = functools.partial(
        pl.kernel,
        out_type=[
            jax.ShapeDtypeStruct((NW * OPAD,), jnp.float32),
            jax.ShapeDtypeStruct((NW * OPAD, BINS), jnp.float32),
        ],
        mesh=plsc.VectorSubcoreMesh(core_axis_name="c", subcore_axis_name="s"),
        compiler_params=pltpu.CompilerParams(needs_layout_passes=False),
        scratch_types=[
            pltpu.VMEM((RPAD,), jnp.int32),
            pltpu.VMEM((RPAD,), jnp.int32),
            pltpu.VMEM((RPAD,), jnp.int32),
            pltpu.VMEM((RPAD, BINS), jnp.float32),
            pltpu.VMEM((RPAD, BINS), jnp.float32),
            pltpu.VMEM((RPAD,), jnp.float32),
            pltpu.SemaphoreType.DMA,
        ],
    )(_tec_body)
    return run(feat, b1d, y1d, x1d)


def kernel(features_z, pose_indexes):
    feat = features_z.reshape(B * Y * X, BINS)
    pidx = jnp.pad(pose_indexes.reshape(NW, RPW, 3),
                   ((0, 0), (0, RPAD - RPW), (0, 0)))
    b1d = pidx[..., 0].reshape(-1)
    y1d = pidx[..., 1].reshape(-1)
    x1d = pidx[..., 2].reshape(-1)
    poses_pad, probs_pad = _sc_extract(feat, b1d, y1d, x1d)
    poses = poses_pad.reshape(NW, OPAD)[:, :RPW].reshape(B, KP)
    probs = probs_pad.reshape(NW, OPAD, BINS)[:, :RPW].reshape(B, KP, BINS)
    return poses, probs


# speedup vs baseline: 18.7479x; 7.4910x over previous
import functools

import jax
import jax.numpy as jnp
from jax import lax
from jax.experimental import pallas as pl
from jax.experimental.pallas import tpu as pltpu
from jax.experimental.pallas import tpu_sc as plsc

B, Y, X, BINS = 128, 96, 96, 64
KP = 17
Z_SIZE = 1.0
NROWS = B * KP
NC, NS, L = 2, 16, 16
NW = NC * NS
RPW = NROWS // NW
RPAD = 80
OPAD = 72


def _tec_body(feat_hbm, b_hbm, y_hbm, x_hbm, pose_out, prob_out,
              b_v, y_v, x_v, eidx_v, rows_v, probs_v, poses_v, sem):
    wid = lax.axis_index("s") * NC + lax.axis_index("c")
    base = wid * RPAD
    pltpu.sync_copy(b_hbm.at[pl.ds(base, RPAD)], b_v)
    pltpu.sync_copy(y_hbm.at[pl.ds(base, RPAD)], y_v)
    pltpu.sync_copy(x_hbm.at[pl.ds(base, RPAD)], x_v)

    lane = lax.iota(jnp.int32, L)
    zoffs = [(lane + kz * L) * B for kz in range(BINS // L)]
    for c in range(RPAD // L):
        bi = b_v[pl.ds(c * L, L)]
        yi = y_v[pl.ds(c * L, L)]
        xi = x_v[pl.ds(c * L, L)]
        ei = (yi * X + xi) * (BINS * B) + bi
        for j in range(L):
            ebase = lax.broadcast_in_dim(ei[j], (L,), ())
            for kz in range(BINS // L):
                eidx_v[c * L + j, pl.ds(kz * L, L)] = ebase + zoffs[kz]

    copies = [pltpu.async_copy(feat_hbm.at[eidx_v.at[i]], rows_v.at[i], sem)
              for i in range(RPAD)]
    for cp in copies:
        cp.wait()

    locs = [(lax.iota(jnp.int32, L) + k * L).astype(jnp.float32)
            * (2.0 * Z_SIZE / (BINS - 1)) - Z_SIZE for k in range(BINS // L)]

    def group_body(g, carry):
        acc = jnp.zeros((L,), jnp.float32)
        for j in range(L):
            r = g * L + j
            vs = [rows_v[r, pl.ds(k * L, L)] for k in range(BINS // L)]
            m = jnp.max(jnp.maximum(jnp.maximum(vs[0], vs[1]),
                                    jnp.maximum(vs[2], vs[3])))
            es = [jnp.exp(v - m) for v in vs]
            s = jnp.sum(es[0] + es[1] + es[2] + es[3])
            w = jnp.sum(es[0] * locs[0] + es[1] * locs[1]
                        + es[2] * locs[2] + es[3] * locs[3])
            invv = 1.0 / lax.broadcast_in_dim(s, (L,), ())
            for k in range(BINS // L):
                probs_v[r, pl.ds(k * L, L)] = es[k] * invv
            acc = jnp.where(lane == j,
                            lax.broadcast_in_dim(w, (L,), ()) * invv, acc)
        poses_v[pl.ds(pl.multiple_of(g * L, L), L)] = acc
        return carry

    lax.fori_loop(0, RPAD // L, group_body, 0)

    pltpu.sync_copy(poses_v.at[pl.ds(0, OPAD)],
                    pose_out.at[pl.ds(wid * OPAD, OPAD)])
    pltpu.sync_copy(probs_v.at[pl.ds(0, OPAD)],
                    prob_out.at[pl.ds(wid * OPAD, OPAD)])


@functools.partial(jax.jit)
def _sc_extract(feat, b1d, y1d, x1d):
    run = functools.partial(
        pl.kernel,
        out_type=[
            jax.ShapeDtypeStruct((NW * OPAD,), jnp.float32),
            jax.ShapeDtypeStruct((NW * OPAD, BINS), jnp.float32),
        ],
        mesh=plsc.VectorSubcoreMesh(core_axis_name="c", subcore_axis_name="s"),
        compiler_params=pltpu.CompilerParams(
            needs_layout_passes=False, use_tc_tiling_on_sc=False),
        scratch_types=[
            pltpu.VMEM((RPAD,), jnp.int32),
            pltpu.VMEM((RPAD,), jnp.int32),
            pltpu.VMEM((RPAD,), jnp.int32),
            pltpu.VMEM((RPAD, BINS), jnp.int32),
            pltpu.VMEM((RPAD, BINS), jnp.float32),
            pltpu.VMEM((RPAD, BINS), jnp.float32),
            pltpu.VMEM((RPAD,), jnp.float32),
            pltpu.SemaphoreType.DMA,
        ],
    )(_tec_body)
    return run(feat, b1d, y1d, x1d)


def kernel(features_z, pose_indexes):
    feat = features_z.transpose(1, 2, 3, 0).reshape(-1)
    pidx = jnp.pad(pose_indexes.reshape(NW, RPW, 3),
                   ((0, 0), (0, RPAD - RPW), (0, 0)))
    b1d = pidx[..., 0].reshape(-1)
    y1d = pidx[..., 1].reshape(-1)
    x1d = pidx[..., 2].reshape(-1)
    poses_pad, probs_pad = _sc_extract(feat, b1d, y1d, x1d)
    poses = poses_pad.reshape(NW, OPAD)[:, :RPW].reshape(B, KP)
    probs = probs_pad.reshape(NW, OPAD, BINS)[:, :RPW].reshape(B, KP, BINS)
    return poses, probs


# speedup vs baseline: 19.2169x; 1.0250x over previous
import functools

import jax
import jax.numpy as jnp
from jax import lax
from jax.experimental import pallas as pl
from jax.experimental.pallas import tpu as pltpu
from jax.experimental.pallas import tpu_sc as plsc

B, Y, X, BINS = 128, 96, 96, 64
KP = 17
Z_SIZE = 1.0
NROWS = B * KP
NC, NS, L = 2, 16, 16
NW = NC * NS
RPW = NROWS // NW
RPAD = 80
NIDX = NW * RPAD


def _tec_body(feat_hbm, idx_hbm, pose_out, prob_out,
              b_v, y_v, x_v, eidx_v, rows_v, probs_v, poses_v, sem):
    wid = lax.axis_index("s") * NC + lax.axis_index("c")
    base = wid * RPAD
    pltpu.sync_copy(idx_hbm.at[pl.ds(base, RPAD)], b_v)
    pltpu.sync_copy(idx_hbm.at[pl.ds(NIDX + base, RPAD)], y_v)
    pltpu.sync_copy(idx_hbm.at[pl.ds(2 * NIDX + base, RPAD)], x_v)

    lane = lax.iota(jnp.int32, L)
    zoffs = [(lane + kz * L) * B for kz in range(BINS // L)]
    for c in range(RPAD // L):
        bi = b_v[pl.ds(c * L, L)]
        yi = y_v[pl.ds(c * L, L)]
        xi = x_v[pl.ds(c * L, L)]
        ei = (yi * X + xi) * (BINS * B) + bi
        for j in range(L):
            ebase = lax.broadcast_in_dim(ei[j], (L,), ())
            for kz in range(BINS // L):
                eidx_v[pl.ds((c * L + j) * BINS + kz * L, L)] = (
                    ebase + zoffs[kz])

    pltpu.async_copy(feat_hbm.at[eidx_v], rows_v, sem).wait()

    locs = [(lax.iota(jnp.int32, L) + k * L).astype(jnp.float32)
            * (2.0 * Z_SIZE / (BINS - 1)) - Z_SIZE for k in range(BINS // L)]

    def group_body(g, carry):
        acc = jnp.zeros((L,), jnp.float32)
        for j in range(L):
            r = g * L + j
            vs = [rows_v[pl.ds(r * BINS + k * L, L)]
                  for k in range(BINS // L)]
            m = jnp.max(jnp.maximum(jnp.maximum(vs[0], vs[1]),
                                    jnp.maximum(vs[2], vs[3])))
            es = [jnp.exp(v - m) for v in vs]
            s = jnp.sum(es[0] + es[1] + es[2] + es[3])
            w = jnp.sum(es[0] * locs[0] + es[1] * locs[1]
                        + es[2] * locs[2] + es[3] * locs[3])
            invv = 1.0 / lax.broadcast_in_dim(s, (L,), ())
            for k in range(BINS // L):
                probs_v[r, pl.ds(k * L, L)] = es[k] * invv
            acc = jnp.where(lane == j,
                            lax.broadcast_in_dim(w, (L,), ()) * invv, acc)
        poses_v[pl.ds(pl.multiple_of(g * L, L), L)] = acc
        return carry

    lax.fori_loop(0, RPAD // L, group_body, 0)

    pltpu.sync_copy(poses_v.at[pl.ds(0, RPW)], pose_out.at[wid])
    pltpu.sync_copy(probs_v.at[pl.ds(0, RPW)],
                    prob_out.at[pl.ds(wid * RPW, RPW)])


@functools.partial(jax.jit)
def _sc_extract(feat, idx_flat):
    run = functools.partial(
        pl.kernel,
        out_type=[
            jax.ShapeDtypeStruct((NW, RPW), jnp.float32),
            jax.ShapeDtypeStruct((NROWS, BINS), jnp.float32),
        ],
        mesh=plsc.VectorSubcoreMesh(core_axis_name="c", subcore_axis_name="s"),
        compiler_params=pltpu.CompilerParams(
            needs_layout_passes=False, use_tc_tiling_on_sc=False),
        scratch_types=[
            pltpu.VMEM((RPAD,), jnp.int32),
            pltpu.VMEM((RPAD,), jnp.int32),
            pltpu.VMEM((RPAD,), jnp.int32),
            pltpu.VMEM((RPAD * BINS,), jnp.int32),
            pltpu.VMEM((RPAD * BINS,), jnp.float32),
            pltpu.VMEM((RPAD, BINS), jnp.float32),
            pltpu.VMEM((RPAD,), jnp.float32),
            pltpu.SemaphoreType.DMA,
        ],
    )(_tec_body)
    return run(feat, idx_flat)


def kernel(features_z, pose_indexes):
    feat = features_z.transpose(1, 2, 3, 0).reshape(-1)
    pidx = jnp.pad(pose_indexes.reshape(NW, RPW, 3),
                   ((0, 0), (0, RPAD - RPW), (0, 0)))
    idx_flat = pidx.transpose(2, 0, 1).reshape(-1)
    poses, probs = _sc_extract(feat, idx_flat)
    return poses.reshape(B, KP), probs.reshape(B, KP, BINS)


# speedup vs baseline: 19.7271x; 1.0266x over previous
import functools

import jax
import jax.numpy as jnp
from jax import lax
from jax.experimental import pallas as pl
from jax.experimental.pallas import tpu as pltpu
from jax.experimental.pallas import tpu_sc as plsc

B, Y, X, BINS = 128, 96, 96, 64
KP = 17
Z_SIZE = 1.0
NROWS = B * KP
NC, NS, L = 2, 16, 16
NW = NC * NS
RPW = NROWS // NW
RPAD = 80
GRP = RPAD // L
ZC = BINS // L


def _tec_body(feat_hbm, idx_hbm, pose_out, prob_out,
              bxy_v, eidx_v, rows_v, probs_v, poses_v, sem):
    wid = lax.axis_index("s") * NC + lax.axis_index("c")
    pltpu.sync_copy(idx_hbm.at[pl.ds(wid * (3 * RPAD), 3 * RPAD)], bxy_v)

    lane = lax.iota(jnp.int32, L)
    zoffs = [(lane + kz * L) * B for kz in range(ZC)]

    def eidx_body(c, carry):
        c16 = pl.multiple_of(c * L, L)
        bi = bxy_v[pl.ds(c16, L)]
        yi = bxy_v[pl.ds(RPAD + c16, L)]
        xi = bxy_v[pl.ds(2 * RPAD + c16, L)]
        ei = (yi * X + xi) * (BINS * B) + bi
        for j in range(L):
            ebase = lax.broadcast_in_dim(ei[j], (L,), ())
            for kz in range(ZC):
                eidx_v[pl.ds((c16 + j) * BINS + kz * L, L)] = ebase + zoffs[kz]
        return carry

    lax.fori_loop(0, GRP, eidx_body, 0)

    pltpu.async_copy(feat_hbm.at[eidx_v], rows_v, sem).wait()

    locs = [(lax.iota(jnp.int32, L) + k * L).astype(jnp.float32)
            * (2.0 * Z_SIZE / (BINS - 1)) - Z_SIZE for k in range(ZC)]

    def group_body(g, carry):
        g16 = pl.multiple_of(g * L, L)
        acc = jnp.zeros((L,), jnp.float32)
        for j in range(L):
            roff = (g16 + j) * BINS
            vs = [rows_v[pl.ds(roff + k * L, L)] for k in range(ZC)]
            m = jnp.max(jnp.maximum(jnp.maximum(vs[0], vs[1]),
                                    jnp.maximum(vs[2], vs[3])))
            es = [jnp.exp(v - m) for v in vs]
            s = jnp.sum(es[0] + es[1] + es[2] + es[3])
            w = jnp.sum(es[0] * locs[0] + es[1] * locs[1]
                        + es[2] * locs[2] + es[3] * locs[3])
            invv = 1.0 / lax.broadcast_in_dim(s, (L,), ())
            for k in range(ZC):
                probs_v[g16 + j, pl.ds(k * L, L)] = es[k] * invv
            acc = jnp.where(lane == j,
                            lax.broadcast_in_dim(w, (L,), ()) * invv, acc)
        poses_v[pl.ds(g16, L)] = acc
        return carry

    lax.fori_loop(0, GRP, group_body, 0)

    pltpu.sync_copy(poses_v.at[pl.ds(0, RPW)], pose_out.at[wid])
    pltpu.sync_copy(probs_v.at[pl.ds(0, RPW)],
                    prob_out.at[pl.ds(wid * RPW, RPW)])


@functools.partial(jax.jit)
def _sc_extract(feat, idx_flat):
    run = functools.partial(
        pl.kernel,
        out_type=[
            jax.ShapeDtypeStruct((NW, RPW), jnp.float32),
            jax.ShapeDtypeStruct((NROWS, BINS), jnp.float32),
        ],
        mesh=plsc.VectorSubcoreMesh(core_axis_name="c", subcore_axis_name="s"),
        compiler_params=pltpu.CompilerParams(
            needs_layout_passes=False, use_tc_tiling_on_sc=False),
        scratch_types=[
            pltpu.VMEM((3 * RPAD,), jnp.int32),
            pltpu.VMEM((RPAD * BINS,), jnp.int32),
            pltpu.VMEM((RPAD * BINS,), jnp.float32),
            pltpu.VMEM((RPAD, BINS), jnp.float32),
            pltpu.VMEM((RPAD,), jnp.float32),
            pltpu.SemaphoreType.DMA,
        ],
    )(_tec_body)
    return run(feat, idx_flat)


def kernel(features_z, pose_indexes):
    feat = features_z.transpose(1, 2, 3, 0).reshape(-1)
    pidx = jnp.pad(pose_indexes.reshape(NW, RPW, 3),
                   ((0, 0), (0, RPAD - RPW), (0, 0)))
    idx_flat = pidx.transpose(0, 2, 1).reshape(-1)
    poses, probs = _sc_extract(feat, idx_flat)
    return poses.reshape(B, KP), probs.reshape(B, KP, BINS)
